# baseline (speedup 1.0000x reference)
import functools

import jax
import jax.numpy as jnp
from jax import lax
from jax.experimental import pallas as pl
from jax.experimental.pallas import tpu as pltpu
from jax.experimental.pallas import tpu_sc as plsc

N = 4096
M = 1024
ETA = 0.1
MU_WTS = 2.5

LW_BLK = 128
GB = 8
IDX_CAP = N + 128


def _sc_compact_body(f_hbm, idx_hbm, cnt_hbm, f_v, idx_v, cnt_v):
    cid = lax.axis_index("c")
    sid = lax.axis_index("s")
    wid = sid * 2 + cid

    @pl.when(wid == 0)
    def _():
        pltpu.sync_copy(f_hbm, f_v)
        iota = lax.broadcasted_iota(jnp.int32, (16,), 0)
        zeros16 = jnp.zeros((16,), jnp.int32)

        def compact_body(c, cnt):
            fv = f_v[pl.ds(c * 16, 16)]
            mi = fv.astype(jnp.int32)
            pos = plsc.cumsum(mi)
            plsc.store_scatter(idx_v, [pos + (cnt - 1)], iota + c * 16,
                               mask=fv > 0.0)
            return cnt + jnp.sum(mi)

        cnt = lax.fori_loop(0, N // 16, compact_body, jnp.int32(0))
        idx_v[pl.ds(cnt, 16)] = zeros16
        cnt_v[...] = zeros16 + cnt
        pltpu.sync_copy(idx_v, idx_hbm)
        pltpu.sync_copy(cnt_v, cnt_hbm)


@functools.cache
def _sc_compact():
    return pl.kernel(
        _sc_compact_body,
        out_type=[
            jax.ShapeDtypeStruct((IDX_CAP,), jnp.int32),
            jax.ShapeDtypeStruct((16,), jnp.int32),
        ],
        mesh=plsc.VectorSubcoreMesh(core_axis_name="c",
                                    subcore_axis_name="s",
                                    num_cores=2, num_subcores=16),
        compiler_params=pltpu.CompilerParams(needs_layout_passes=False),
        scratch_types=[
            pltpu.VMEM((N,), jnp.float32),
            pltpu.VMEM((IDX_CAP,), jnp.int32),
            pltpu.VMEM((16,), jnp.int32),
        ],
    )


def _gather_body(idx_ref, cnt_ref, x_ref, thr_ref, w_any, nf_ref,
                 buf0, buf1, acc_ref, sem0, sem1):
    cnt = cnt_ref[0]
    nb = (cnt + GB - 1) // GB

    def start_batch(b, buf, sem):
        for r in range(GB):
            j = idx_ref[b * GB + r]
            pltpu.make_async_copy(w_any.at[pl.ds(j, 1)],
                                  buf.at[pl.ds(r, 1)], sem).start()

    def wait_batch(buf, sem):
        for r in range(GB):
            pltpu.make_async_copy(w_any.at[pl.ds(0, 1)],
                                  buf.at[pl.ds(r, 1)], sem).wait()

    acc_ref[...] = jnp.zeros((GB, N), jnp.float32)

    @pl.when(nb > 0)
    def _():
        start_batch(0, buf0, sem0)

    def bbody(b, carry):
        even = lax.rem(b, 2) == 0

        @pl.when(b + 1 < nb)
        def _():
            @pl.when(even)
            def _():
                start_batch(b + 1, buf1, sem1)

            @pl.when(jnp.logical_not(even))
            def _():
                start_batch(b + 1, buf0, sem0)

        rows = lax.broadcasted_iota(jnp.int32, (GB, 1), 0) + b * GB
        wmask = jnp.where(rows < cnt, 1.0, 0.0).astype(jnp.float32)

        @pl.when(even)
        def _():
            wait_batch(buf0, sem0)
            acc_ref[...] += buf0[...] * wmask

        @pl.when(jnp.logical_not(even))
        def _():
            wait_batch(buf1, sem1)
            acc_ref[...] += buf1[...] * wmask

        return carry

    lax.fori_loop(0, nb, bbody, jnp.int32(0))

    node = jnp.sum(acc_ref[...], axis=0, keepdims=True)
    nf_ref[...] = (node + x_ref[...] > thr_ref[...]).astype(jnp.float32)


def _phase2_body(nf_ref, w_ref, thr_ref, wout_ref, act_ref, maxv_ref,
                 maxi_ref, smax, sidx):
    i = pl.program_id(0)
    w = w_ref[...]
    wout_ref[...] = w
    wb = w.astype(jnp.bfloat16).astype(jnp.float32)
    a = lax.dot_general(wb, nf_ref[...], (((1,), (1,)), ((), ())),
                        preferred_element_type=jnp.float32)
    lgn_act = jnp.maximum(a, 0.0)
    act_ref[...] = lgn_act
    actv = jnp.maximum(lgn_act - thr_ref[...], 0.0)
    bmax = jnp.max(actv)
    iota = lax.broadcasted_iota(jnp.int32, (LW_BLK, 1), 0)
    bidx = jnp.min(jnp.where(actv == bmax, iota, 2 ** 30)) + i * LW_BLK

    @pl.when(i == 0)
    def _():
        smax[0] = bmax
        sidx[0] = bidx

    @pl.when(i > 0)
    def _():
        better = bmax > smax[0]
        smax[0] = jnp.where(better, bmax, smax[0])
        sidx[0] = jnp.where(better, bidx, sidx[0])

    @pl.when(i == pl.num_programs(0) - 1)
    def _():
        maxv_ref[0, 0] = smax[0]
        maxi_ref[0, 0] = sidx[0]


def _phase3_body(maxi_ref, maxv_ref, nf_ref, thr_ref, w_any, wout_any,
                 throut_ref, row_ref, sem):
    idx = maxi_ref[0, 0]
    maxv = maxv_ref[0, 0]
    fired = maxv > 0.0
    iota = lax.broadcasted_iota(jnp.int32, (M, 1), 0)
    bump = jnp.where((iota == idx) & fired, 0.005 * maxv, 0.0)
    throut_ref[...] = thr_ref[...] + bump

    @pl.when(fired)
    def _():
        cp_in = pltpu.make_async_copy(w_any.at[pl.ds(idx, 1)], row_ref, sem)
        cp_in.start()
        cp_in.wait()
        w_new = row_ref[...] + (ETA * maxv) * nf_ref[...]
        mean = jnp.sum(w_new) / float(N)
        row_ref[...] = w_new / mean * MU_WTS
        cp_out = pltpu.make_async_copy(row_ref, wout_any.at[pl.ds(idx, 1)],
                                       sem)
        cp_out.start()
        cp_out.wait()


@jax.jit
def kernel(x, is_firing, retina_weights, lgn_weights, lgn_threshold,
           node_threshold):
    x_row = x.reshape(1, N)
    nthr_row = node_threshold.reshape(1, N)
    lthr_col = lgn_threshold.reshape(M, 1)

    idx, cnt = _sc_compact()(is_firing)

    nf_row = pl.pallas_call(
        _gather_body,
        grid=(1,),
        in_specs=[
            pl.BlockSpec(memory_space=pltpu.SMEM),
            pl.BlockSpec(memory_space=pltpu.SMEM),
            pl.BlockSpec((1, N), lambda i: (0, 0)),
            pl.BlockSpec((1, N), lambda i: (0, 0)),
            pl.BlockSpec(memory_space=pl.ANY),
        ],
        out_specs=pl.BlockSpec((1, N), lambda i: (0, 0)),
        out_shape=jax.ShapeDtypeStruct((1, N), jnp.float32),
        scratch_shapes=[
            pltpu.VMEM((GB, N), jnp.float32),
            pltpu.VMEM((GB, N), jnp.float32),
            pltpu.VMEM((GB, N), jnp.float32),
            pltpu.SemaphoreType.DMA,
            pltpu.SemaphoreType.DMA,
        ],
    )(idx, cnt, x_row, nthr_row, retina_weights)

    wcopy, lgn_act_col, maxv, maxi = pl.pallas_call(
        _phase2_body,
        grid=(M // LW_BLK,),
        in_specs=[
            pl.BlockSpec((1, N), lambda i: (0, 0)),
            pl.BlockSpec((LW_BLK, N), lambda i: (i, 0)),
            pl.BlockSpec((LW_BLK, 1), lambda i: (i, 0)),
        ],
        out_specs=[
            pl.BlockSpec((LW_BLK, N), lambda i: (i, 0)),
            pl.BlockSpec((LW_BLK, 1), lambda i: (i, 0)),
            pl.BlockSpec(memory_space=pltpu.SMEM),
            pl.BlockSpec(memory_space=pltpu.SMEM),
        ],
        out_shape=[
            jax.ShapeDtypeStruct((M, N), jnp.float32),
            jax.ShapeDtypeStruct((M, 1), jnp.float32),
            jax.ShapeDtypeStruct((1, 1), jnp.float32),
            jax.ShapeDtypeStruct((1, 1), jnp.int32),
        ],
        scratch_shapes=[
            pltpu.SMEM((1,), jnp.float32),
            pltpu.SMEM((1,), jnp.int32),
        ],
    )(nf_row, lgn_weights, lthr_col)

    new_w, new_thr_col = pl.pallas_call(
        _phase3_body,
        grid=(1,),
        in_specs=[
            pl.BlockSpec(memory_space=pltpu.SMEM),
            pl.BlockSpec(memory_space=pltpu.SMEM),
            pl.BlockSpec((1, N), lambda i: (0, 0)),
            pl.BlockSpec((M, 1), lambda i: (0, 0)),
            pl.BlockSpec(memory_space=pl.ANY),
        ],
        out_specs=[
            pl.BlockSpec(memory_space=pl.ANY),
            pl.BlockSpec((M, 1), lambda i: (0, 0)),
        ],
        out_shape=[
            jax.ShapeDtypeStruct((M, N), jnp.float32),
            jax.ShapeDtypeStruct((M, 1), jnp.float32),
        ],
        scratch_shapes=[
            pltpu.VMEM((1, N), jnp.float32),
            pltpu.SemaphoreType.DMA,
        ],
        input_output_aliases={4: 0},
    )(maxi, maxv, nf_row, lthr_col, wcopy)

    return (lgn_act_col.reshape(M), nf_row.reshape(N), new_w,
            new_thr_col.reshape(M))


# speedup vs baseline: 2.0508x; 2.0508x over previous
import functools

import jax
import jax.numpy as jnp
from jax import lax
from jax.experimental import pallas as pl
from jax.experimental.pallas import tpu as pltpu

N = 4096
M = 1024
ETA = 0.1
MU_WTS = 2.5

RW_BLK = 256
LW_BLK = 128


def _phase1_body(f_ref, x_ref, thr_ref, w_ref, nf_ref):
    nx = lax.dot_general(f_ref[...], w_ref[...],
                         (((1,), (0,)), ((), ())),
                         preferred_element_type=jnp.float32)
    nf_ref[...] = (nx + x_ref[...] > thr_ref[...]).astype(jnp.float32)


def _phase2_body(nf_ref, w_ref, thr_ref, wout_ref, act_ref, maxv_ref,
                 maxi_ref, smax, sidx):
    i = pl.program_id(0)
    w = w_ref[...]
    wout_ref[...] = w
    wb = w.astype(jnp.bfloat16).astype(jnp.float32)
    a = lax.dot_general(wb, nf_ref[...], (((1,), (1,)), ((), ())),
                        preferred_element_type=jnp.float32)
    lgn_act = jnp.maximum(a, 0.0)
    act_ref[...] = lgn_act
    actv = jnp.maximum(lgn_act - thr_ref[...], 0.0)
    bmax = jnp.max(actv)
    iota = lax.broadcasted_iota(jnp.int32, (LW_BLK, 1), 0)
    bidx = jnp.min(jnp.where(actv == bmax, iota, 2 ** 30)) + i * LW_BLK

    @pl.when(i == 0)
    def _():
        smax[0] = bmax
        sidx[0] = bidx

    @pl.when(i > 0)
    def _():
        better = bmax > smax[0]
        smax[0] = jnp.where(better, bmax, smax[0])
        sidx[0] = jnp.where(better, bidx, sidx[0])

    @pl.when(i == pl.num_programs(0) - 1)
    def _():
        maxv_ref[0, 0] = smax[0]
        maxi_ref[0, 0] = sidx[0]


def _phase3_body(maxi_ref, maxv_ref, nf_ref, thr_ref, w_any, wout_any,
                 throut_ref, row_ref, sem):
    idx = maxi_ref[0, 0]
    maxv = maxv_ref[0, 0]
    fired = maxv > 0.0
    iota = lax.broadcasted_iota(jnp.int32, (M, 1), 0)
    bump = jnp.where((iota == idx) & fired, 0.005 * maxv, 0.0)
    throut_ref[...] = thr_ref[...] + bump

    @pl.when(fired)
    def _():
        cp_in = pltpu.make_async_copy(w_any.at[pl.ds(idx, 1)], row_ref, sem)
        cp_in.start()
        cp_in.wait()
        w_new = row_ref[...] + (ETA * maxv) * nf_ref[...]
        mean = jnp.sum(w_new) / float(N)
        row_ref[...] = w_new / mean * MU_WTS
        cp_out = pltpu.make_async_copy(row_ref, wout_any.at[pl.ds(idx, 1)],
                                       sem)
        cp_out.start()
        cp_out.wait()


@jax.jit
def kernel(x, is_firing, retina_weights, lgn_weights, lgn_threshold,
           node_threshold):
    f_row = is_firing.reshape(1, N)
    x_row = x.reshape(1, N)
    nthr_row = node_threshold.reshape(1, N)
    lthr_col = lgn_threshold.reshape(M, 1)

    nf_row = pl.pallas_call(
        _phase1_body,
        grid=(N // RW_BLK,),
        in_specs=[
            pl.BlockSpec((1, N), lambda i: (0, 0)),
            pl.BlockSpec((1, RW_BLK), lambda i: (0, i)),
            pl.BlockSpec((1, RW_BLK), lambda i: (0, i)),
            pl.BlockSpec((N, RW_BLK), lambda i: (0, i)),
        ],
        out_specs=pl.BlockSpec((1, RW_BLK), lambda i: (0, i)),
        out_shape=jax.ShapeDtypeStruct((1, N), jnp.float32),
    )(f_row, x_row, nthr_row, retina_weights)

    wcopy, lgn_act_col, maxv, maxi = pl.pallas_call(
        _phase2_body,
        grid=(M // LW_BLK,),
        in_specs=[
            pl.BlockSpec((1, N), lambda i: (0, 0)),
            pl.BlockSpec((LW_BLK, N), lambda i: (i, 0)),
            pl.BlockSpec((LW_BLK, 1), lambda i: (i, 0)),
        ],
        out_specs=[
            pl.BlockSpec((LW_BLK, N), lambda i: (i, 0)),
            pl.BlockSpec((LW_BLK, 1), lambda i: (i, 0)),
            pl.BlockSpec(memory_space=pltpu.SMEM),
            pl.BlockSpec(memory_space=pltpu.SMEM),
        ],
        out_shape=[
            jax.ShapeDtypeStruct((M, N), jnp.float32),
            jax.ShapeDtypeStruct((M, 1), jnp.float32),
            jax.ShapeDtypeStruct((1, 1), jnp.float32),
            jax.ShapeDtypeStruct((1, 1), jnp.int32),
        ],
        scratch_shapes=[
            pltpu.SMEM((1,), jnp.float32),
            pltpu.SMEM((1,), jnp.int32),
        ],
    )(nf_row, lgn_weights, lthr_col)

    new_w, new_thr_col = pl.pallas_call(
        _phase3_body,
        grid=(1,),
        in_specs=[
            pl.BlockSpec(memory_space=pltpu.SMEM),
            pl.BlockSpec(memory_space=pltpu.SMEM),
            pl.BlockSpec((1, N), lambda i: (0, 0)),
            pl.BlockSpec((M, 1), lambda i: (0, 0)),
            pl.BlockSpec(memory_space=pl.ANY),
        ],
        out_specs=[
            pl.BlockSpec(memory_space=pl.ANY),
            pl.BlockSpec((M, 1), lambda i: (0, 0)),
        ],
        out_shape=[
            jax.ShapeDtypeStruct((M, N), jnp.float32),
            jax.ShapeDtypeStruct((M, 1), jnp.float32),
        ],
        scratch_shapes=[
            pltpu.VMEM((1, N), jnp.float32),
            pltpu.SemaphoreType.DMA,
        ],
        input_output_aliases={4: 0},
    )(maxi, maxv, nf_row, lthr_col, wcopy)

    return (lgn_act_col.reshape(M), nf_row.reshape(N), new_w,
            new_thr_col.reshape(M))


# speedup vs baseline: 2.0975x; 1.0228x over previous
import functools

import jax
import jax.numpy as jnp
from jax import lax
from jax.experimental import pallas as pl
from jax.experimental.pallas import tpu as pltpu

N = 4096
M = 1024
ETA = 0.1
MU_WTS = 2.5

RW_BLK = 1024
LW_BLK = 128


def _phase1_body(f_ref, x_ref, thr_ref, w_ref, nf_ref):
    nx = lax.dot_general(f_ref[...], w_ref[...],
                         (((1,), (0,)), ((), ())),
                         preferred_element_type=jnp.float32)
    nf_ref[...] = (nx + x_ref[...] > thr_ref[...]).astype(jnp.float32)


def _phase2_body(nf_ref, w_ref, thr_ref, wout_ref, act_ref, maxv_ref,
                 maxi_ref, smax, sidx):
    i = pl.program_id(0)
    w = w_ref[...]
    wout_ref[...] = w
    wb = w.astype(jnp.bfloat16).astype(jnp.float32)
    a = lax.dot_general(wb, nf_ref[...], (((1,), (1,)), ((), ())),
                        preferred_element_type=jnp.float32)
    lgn_act = jnp.maximum(a, 0.0)
    act_ref[...] = lgn_act
    actv = jnp.maximum(lgn_act - thr_ref[...], 0.0)
    bmax = jnp.max(actv)
    iota = lax.broadcasted_iota(jnp.int32, (LW_BLK, 1), 0)
    bidx = jnp.min(jnp.where(actv == bmax, iota, 2 ** 30)) + i * LW_BLK

    @pl.when(i == 0)
    def _():
        smax[0] = bmax
        sidx[0] = bidx

    @pl.when(i > 0)
    def _():
        better = bmax > smax[0]
        smax[0] = jnp.where(better, bmax, smax[0])
        sidx[0] = jnp.where(better, bidx, sidx[0])

    @pl.when(i == pl.num_programs(0) - 1)
    def _():
        maxv_ref[0, 0] = smax[0]
        maxi_ref[0, 0] = sidx[0]


def _phase3_body(maxi_ref, maxv_ref, nf_ref, thr_ref, w_any, wout_any,
                 throut_ref, row_ref, sem):
    idx = maxi_ref[0, 0]
    maxv = maxv_ref[0, 0]
    fired = maxv > 0.0
    iota = lax.broadcasted_iota(jnp.int32, (M, 1), 0)
    bump = jnp.where((iota == idx) & fired, 0.005 * maxv, 0.0)
    throut_ref[...] = thr_ref[...] + bump

    @pl.when(fired)
    def _():
        cp_in = pltpu.make_async_copy(w_any.at[pl.ds(idx, 1)], row_ref, sem)
        cp_in.start()
        cp_in.wait()
        w_new = row_ref[...] + (ETA * maxv) * nf_ref[...]
        mean = jnp.sum(w_new) / float(N)
        row_ref[...] = w_new / mean * MU_WTS
        cp_out = pltpu.make_async_copy(row_ref, wout_any.at[pl.ds(idx, 1)],
                                       sem)
        cp_out.start()
        cp_out.wait()


@jax.jit
def kernel(x, is_firing, retina_weights, lgn_weights, lgn_threshold,
           node_threshold):
    f_row = is_firing.reshape(1, N)
    x_row = x.reshape(1, N)
    nthr_row = node_threshold.reshape(1, N)
    lthr_col = lgn_threshold.reshape(M, 1)

    nf_row = pl.pallas_call(
        _phase1_body,
        grid=(N // RW_BLK,),
        in_specs=[
            pl.BlockSpec((1, N), lambda i: (0, 0)),
            pl.BlockSpec((1, RW_BLK), lambda i: (0, i)),
            pl.BlockSpec((1, RW_BLK), lambda i: (0, i)),
            pl.BlockSpec((N, RW_BLK), lambda i: (0, i)),
        ],
        out_specs=pl.BlockSpec((1, RW_BLK), lambda i: (0, i)),
        out_shape=jax.ShapeDtypeStruct((1, N), jnp.float32),
    )(f_row, x_row, nthr_row, retina_weights)

    wcopy, lgn_act_col, maxv, maxi = pl.pallas_call(
        _phase2_body,
        grid=(M // LW_BLK,),
        in_specs=[
            pl.BlockSpec((1, N), lambda i: (0, 0)),
            pl.BlockSpec((LW_BLK, N), lambda i: (i, 0)),
            pl.BlockSpec((LW_BLK, 1), lambda i: (i, 0)),
        ],
        out_specs=[
            pl.BlockSpec((LW_BLK, N), lambda i: (i, 0)),
            pl.BlockSpec((LW_BLK, 1), lambda i: (i, 0)),
            pl.BlockSpec(memory_space=pltpu.SMEM),
            pl.BlockSpec(memory_space=pltpu.SMEM),
        ],
        out_shape=[
            jax.ShapeDtypeStruct((M, N), jnp.float32),
            jax.ShapeDtypeStruct((M, 1), jnp.float32),
            jax.ShapeDtypeStruct((1, 1), jnp.float32),
            jax.ShapeDtypeStruct((1, 1), jnp.int32),
        ],
        scratch_shapes=[
            pltpu.SMEM((1,), jnp.float32),
            pltpu.SMEM((1,), jnp.int32),
        ],
    )(nf_row, lgn_weights, lthr_col)

    new_w, new_thr_col = pl.pallas_call(
        _phase3_body,
        grid=(1,),
        in_specs=[
            pl.BlockSpec(memory_space=pltpu.SMEM),
            pl.BlockSpec(memory_space=pltpu.SMEM),
            pl.BlockSpec((1, N), lambda i: (0, 0)),
            pl.BlockSpec((M, 1), lambda i: (0, 0)),
            pl.BlockSpec(memory_space=pl.ANY),
        ],
        out_specs=[
            pl.BlockSpec(memory_space=pl.ANY),
            pl.BlockSpec((M, 1), lambda i: (0, 0)),
        ],
        out_shape=[
            jax.ShapeDtypeStruct((M, N), jnp.float32),
            jax.ShapeDtypeStruct((M, 1), jnp.float32),
        ],
        scratch_shapes=[
            pltpu.VMEM((1, N), jnp.float32),
            pltpu.SemaphoreType.DMA,
        ],
        input_output_aliases={4: 0},
    )(maxi, maxv, nf_row, lthr_col, wcopy)

    return (lgn_act_col.reshape(M), nf_row.reshape(N), new_w,
            new_thr_col.reshape(M))


# speedup vs baseline: 2.1770x; 1.0379x over previous
import functools

import jax
import jax.numpy as jnp
from jax import lax
from jax.experimental import pallas as pl
from jax.experimental.pallas import tpu as pltpu

N = 4096
M = 1024
ETA = 0.1
MU_WTS = 2.5

RW_BLK = 512
LW_BLK = 256


def _phase1_body(f_ref, x_ref, thr_ref, w_ref, nf_ref):
    nx = lax.dot_general(f_ref[...], w_ref[...],
                         (((1,), (0,)), ((), ())),
                         preferred_element_type=jnp.float32)
    nf_ref[...] = (nx + x_ref[...] > thr_ref[...]).astype(jnp.float32)


def _phase2_body(nf_ref, w_ref, thr_ref, wout_ref, act_ref, maxv_ref,
                 maxi_ref, smax, sidx):
    i = pl.program_id(0)
    w = w_ref[...]
    wout_ref[...] = w
    wb = w.astype(jnp.bfloat16).astype(jnp.float32)
    a = lax.dot_general(wb, nf_ref[...], (((1,), (1,)), ((), ())),
                        preferred_element_type=jnp.float32)
    lgn_act = jnp.maximum(a, 0.0)
    act_ref[...] = lgn_act
    actv = jnp.maximum(lgn_act - thr_ref[...], 0.0)
    bmax = jnp.max(actv)
    iota = lax.broadcasted_iota(jnp.int32, (LW_BLK, 1), 0)
    bidx = jnp.min(jnp.where(actv == bmax, iota, 2 ** 30)) + i * LW_BLK

    @pl.when(i == 0)
    def _():
        smax[0] = bmax
        sidx[0] = bidx

    @pl.when(i > 0)
    def _():
        better = bmax > smax[0]
        smax[0] = jnp.where(better, bmax, smax[0])
        sidx[0] = jnp.where(better, bidx, sidx[0])

    @pl.when(i == pl.num_programs(0) - 1)
    def _():
        maxv_ref[0, 0] = smax[0]
        maxi_ref[0, 0] = sidx[0]


def _phase3_body(maxi_ref, maxv_ref, nf_ref, thr_ref, w_any, wout_any,
                 throut_ref, row_ref, sem):
    idx = maxi_ref[0, 0]
    maxv = maxv_ref[0, 0]
    fired = maxv > 0.0
    iota = lax.broadcasted_iota(jnp.int32, (M, 1), 0)
    bump = jnp.where((iota == idx) & fired, 0.005 * maxv, 0.0)
    throut_ref[...] = thr_ref[...] + bump

    @pl.when(fired)
    def _():
        cp_in = pltpu.make_async_copy(w_any.at[pl.ds(idx, 1)], row_ref, sem)
        cp_in.start()
        cp_in.wait()
        w_new = row_ref[...] + (ETA * maxv) * nf_ref[...]
        mean = jnp.sum(w_new) / float(N)
        row_ref[...] = w_new / mean * MU_WTS
        cp_out = pltpu.make_async_copy(row_ref, wout_any.at[pl.ds(idx, 1)],
                                       sem)
        cp_out.start()
        cp_out.wait()


@jax.jit
def kernel(x, is_firing, retina_weights, lgn_weights, lgn_threshold,
           node_threshold):
    f_row = is_firing.reshape(1, N)
    x_row = x.reshape(1, N)
    nthr_row = node_threshold.reshape(1, N)
    lthr_col = lgn_threshold.reshape(M, 1)

    nf_row = pl.pallas_call(
        _phase1_body,
        grid=(N // RW_BLK,),
        in_specs=[
            pl.BlockSpec((1, N), lambda i: (0, 0)),
            pl.BlockSpec((1, RW_BLK), lambda i: (0, i)),
            pl.BlockSpec((1, RW_BLK), lambda i: (0, i)),
            pl.BlockSpec((N, RW_BLK), lambda i: (0, i)),
        ],
        out_specs=pl.BlockSpec((1, RW_BLK), lambda i: (0, i)),
        out_shape=jax.ShapeDtypeStruct((1, N), jnp.float32),
    )(f_row, x_row, nthr_row, retina_weights)

    wcopy, lgn_act_col, maxv, maxi = pl.pallas_call(
        _phase2_body,
        grid=(M // LW_BLK,),
        in_specs=[
            pl.BlockSpec((1, N), lambda i: (0, 0)),
            pl.BlockSpec((LW_BLK, N), lambda i: (i, 0)),
            pl.BlockSpec((LW_BLK, 1), lambda i: (i, 0)),
        ],
        out_specs=[
            pl.BlockSpec((LW_BLK, N), lambda i: (i, 0)),
            pl.BlockSpec((LW_BLK, 1), lambda i: (i, 0)),
            pl.BlockSpec(memory_space=pltpu.SMEM),
            pl.BlockSpec(memory_space=pltpu.SMEM),
        ],
        out_shape=[
            jax.ShapeDtypeStruct((M, N), jnp.float32),
            jax.ShapeDtypeStruct((M, 1), jnp.float32),
            jax.ShapeDtypeStruct((1, 1), jnp.float32),
            jax.ShapeDtypeStruct((1, 1), jnp.int32),
        ],
        scratch_shapes=[
            pltpu.SMEM((1,), jnp.float32),
            pltpu.SMEM((1,), jnp.int32),
        ],
    )(nf_row, lgn_weights, lthr_col)

    new_w, new_thr_col = pl.pallas_call(
        _phase3_body,
        grid=(1,),
        in_specs=[
            pl.BlockSpec(memory_space=pltpu.SMEM),
            pl.BlockSpec(memory_space=pltpu.SMEM),
            pl.BlockSpec((1, N), lambda i: (0, 0)),
            pl.BlockSpec((M, 1), lambda i: (0, 0)),
            pl.BlockSpec(memory_space=pl.ANY),
        ],
        out_specs=[
            pl.BlockSpec(memory_space=pl.ANY),
            pl.BlockSpec((M, 1), lambda i: (0, 0)),
        ],
        out_shape=[
            jax.ShapeDtypeStruct((M, N), jnp.float32),
            jax.ShapeDtypeStruct((M, 1), jnp.float32),
        ],
        scratch_shapes=[
            pltpu.VMEM((1, N), jnp.float32),
            pltpu.SemaphoreType.DMA,
        ],
        input_output_aliases={4: 0},
    )(maxi, maxv, nf_row, lthr_col, wcopy)

    return (lgn_act_col.reshape(M), nf_row.reshape(N), new_w,
            new_thr_col.reshape(M))


# speedup vs baseline: 2.2116x; 1.0159x over previous
import functools

import jax
import jax.numpy as jnp
from jax import lax
from jax.experimental import pallas as pl
from jax.experimental.pallas import tpu as pltpu

N = 4096
M = 1024
ETA = 0.1
MU_WTS = 2.5

RW_BLK = 512
LW_BLK = 512


def _phase1_body(f_ref, x_ref, thr_ref, w_ref, nf_ref):
    nx = lax.dot_general(f_ref[...], w_ref[...],
                         (((1,), (0,)), ((), ())),
                         preferred_element_type=jnp.float32)
    nf_ref[...] = (nx + x_ref[...] > thr_ref[...]).astype(jnp.float32)


def _phase2_body(nf_ref, w_ref, thr_ref, wout_ref, act_ref, maxv_ref,
                 maxi_ref, smax, sidx):
    i = pl.program_id(0)
    w = w_ref[...]
    wout_ref[...] = w
    wb = w.astype(jnp.bfloat16).astype(jnp.float32)
    a = lax.dot_general(wb, nf_ref[...], (((1,), (1,)), ((), ())),
                        preferred_element_type=jnp.float32)
    lgn_act = jnp.maximum(a, 0.0)
    act_ref[...] = lgn_act
    actv = jnp.maximum(lgn_act - thr_ref[...], 0.0)
    bmax = jnp.max(actv)
    iota = lax.broadcasted_iota(jnp.int32, (LW_BLK, 1), 0)
    bidx = jnp.min(jnp.where(actv == bmax, iota, 2 ** 30)) + i * LW_BLK

    @pl.when(i == 0)
    def _():
        smax[0] = bmax
        sidx[0] = bidx

    @pl.when(i > 0)
    def _():
        better = bmax > smax[0]
        smax[0] = jnp.where(better, bmax, smax[0])
        sidx[0] = jnp.where(better, bidx, sidx[0])

    @pl.when(i == pl.num_programs(0) - 1)
    def _():
        maxv_ref[0, 0] = smax[0]
        maxi_ref[0, 0] = sidx[0]


def _phase3_body(maxi_ref, maxv_ref, nf_ref, thr_ref, w_any, wout_any,
                 throut_ref, row_ref, sem):
    idx = maxi_ref[0, 0]
    maxv = maxv_ref[0, 0]
    fired = maxv > 0.0
    iota = lax.broadcasted_iota(jnp.int32, (M, 1), 0)
    bump = jnp.where((iota == idx) & fired, 0.005 * maxv, 0.0)
    throut_ref[...] = thr_ref[...] + bump

    @pl.when(fired)
    def _():
        cp_in = pltpu.make_async_copy(w_any.at[pl.ds(idx, 1)], row_ref, sem)
        cp_in.start()
        cp_in.wait()
        w_new = row_ref[...] + (ETA * maxv) * nf_ref[...]
        mean = jnp.sum(w_new) / float(N)
        row_ref[...] = w_new / mean * MU_WTS
        cp_out = pltpu.make_async_copy(row_ref, wout_any.at[pl.ds(idx, 1)],
                                       sem)
        cp_out.start()
        cp_out.wait()


@jax.jit
def kernel(x, is_firing, retina_weights, lgn_weights, lgn_threshold,
           node_threshold):
    f_row = is_firing.reshape(1, N)
    x_row = x.reshape(1, N)
    nthr_row = node_threshold.reshape(1, N)
    lthr_col = lgn_threshold.reshape(M, 1)

    nf_row = pl.pallas_call(
        _phase1_body,
        grid=(N // RW_BLK,),
        in_specs=[
            pl.BlockSpec((1, N), lambda i: (0, 0)),
            pl.BlockSpec((1, RW_BLK), lambda i: (0, i)),
            pl.BlockSpec((1, RW_BLK), lambda i: (0, i)),
            pl.BlockSpec((N, RW_BLK), lambda i: (0, i)),
        ],
        out_specs=pl.BlockSpec((1, RW_BLK), lambda i: (0, i)),
        out_shape=jax.ShapeDtypeStruct((1, N), jnp.float32),
    )(f_row, x_row, nthr_row, retina_weights)

    wcopy, lgn_act_col, maxv, maxi = pl.pallas_call(
        _phase2_body,
        grid=(M // LW_BLK,),
        in_specs=[
            pl.BlockSpec((1, N), lambda i: (0, 0)),
            pl.BlockSpec((LW_BLK, N), lambda i: (i, 0)),
            pl.BlockSpec((LW_BLK, 1), lambda i: (i, 0)),
        ],
        out_specs=[
            pl.BlockSpec((LW_BLK, N), lambda i: (i, 0)),
            pl.BlockSpec((LW_BLK, 1), lambda i: (i, 0)),
            pl.BlockSpec(memory_space=pltpu.SMEM),
            pl.BlockSpec(memory_space=pltpu.SMEM),
        ],
        out_shape=[
            jax.ShapeDtypeStruct((M, N), jnp.float32),
            jax.ShapeDtypeStruct((M, 1), jnp.float32),
            jax.ShapeDtypeStruct((1, 1), jnp.float32),
            jax.ShapeDtypeStruct((1, 1), jnp.int32),
        ],
        scratch_shapes=[
            pltpu.SMEM((1,), jnp.float32),
            pltpu.SMEM((1,), jnp.int32),
        ],
    )(nf_row, lgn_weights, lthr_col)

    new_w, new_thr_col = pl.pallas_call(
        _phase3_body,
        grid=(1,),
        in_specs=[
            pl.BlockSpec(memory_space=pltpu.SMEM),
            pl.BlockSpec(memory_space=pltpu.SMEM),
            pl.BlockSpec((1, N), lambda i: (0, 0)),
            pl.BlockSpec((M, 1), lambda i: (0, 0)),
            pl.BlockSpec(memory_space=pl.ANY),
        ],
        out_specs=[
            pl.BlockSpec(memory_space=pl.ANY),
            pl.BlockSpec((M, 1), lambda i: (0, 0)),
        ],
        out_shape=[
            jax.ShapeDtypeStruct((M, N), jnp.float32),
            jax.ShapeDtypeStruct((M, 1), jnp.float32),
        ],
        scratch_shapes=[
            pltpu.VMEM((1, N), jnp.float32),
            pltpu.SemaphoreType.DMA,
        ],
        input_output_aliases={4: 0},
    )(maxi, maxv, nf_row, lthr_col, wcopy)

    return (lgn_act_col.reshape(M), nf_row.reshape(N), new_w,
            new_thr_col.reshape(M))
